# baseline (speedup 1.0000x reference)
import functools

import jax
import jax.numpy as jnp
from jax import lax
from jax.experimental import pallas as pl
from jax.experimental.pallas import tpu as pltpu
from jax.experimental.pallas import tpu_sc as plsc

N = 10000
E = 320000
D_FEAT = 128
LATENT = 16
STEPS = 3

E8 = E // 8
N8 = N // 8
BLK_E = 2000

NC, NS = 2, 16
NW = NC * NS
IDX_W = 128
IDX_ROWS = E // IDX_W
NCHUNK = 2
CROWS = IDX_ROWS // NCHUNK
CE8 = E8 // NCHUNK
ROWS_PW = 39
ROUNDS = 3
RPC = ROWS_PW // ROUNDS
TAIL = CROWS - NW * ROWS_PW

_SC_MESH = plsc.VectorSubcoreMesh(core_axis_name="c", subcore_axis_name="s")
_SC_PARAMS = pltpu.CompilerParams(use_tc_tiling_on_sc=False)


def _bd8(W):
    return jnp.kron(jnp.eye(8, dtype=W.dtype), W)


def _t8(b):
    return jnp.tile(b, 8)[None, :]


def _enc_nodes_body(x_ref, w1_ref, b1_ref, w2_ref, b2_ref, ws_ref, wr_ref,
                    nlat_ref, ps_ref, pr_ref):
    h = jnp.maximum(
        jnp.dot(x_ref[...], w1_ref[...], preferred_element_type=jnp.float32)
        + b1_ref[...], 0.0)
    nl = jnp.dot(h, w2_ref[...], preferred_element_type=jnp.float32) + b2_ref[...]
    nlat_ref[...] = nl
    ps_ref[...] = jnp.dot(nl, ws_ref[...], preferred_element_type=jnp.float32)
    pr_ref[...] = jnp.dot(nl, wr_ref[...], preferred_element_type=jnp.float32)


def _enc_nodes(nodes, Wen1, ben1, Wen2, ben2, Ws0, Wr0):
    return pl.pallas_call(
        _enc_nodes_body,
        out_shape=[jax.ShapeDtypeStruct((N8, 128), jnp.float32)] * 3,
    )(nodes.reshape(N8, 8 * D_FEAT), _bd8(Wen1), _t8(ben1), _bd8(Wen2),
      _t8(ben2), _bd8(Ws0), _bd8(Wr0))


def _enc_edges_body(x_ref, w1_ref, b1_ref, w2_ref, b2_ref, out_ref):
    h = jnp.maximum(
        jnp.dot(x_ref[...], w1_ref[...], preferred_element_type=jnp.float32)
        + b1_ref[...], 0.0)
    out_ref[...] = (
        jnp.dot(h, w2_ref[...], preferred_element_type=jnp.float32) + b2_ref[...])


def _enc_edges(c, edges8, Wee1, bee1, Wee2, bee2):
    grid = CE8 // BLK_E
    blk0 = c * grid
    return pl.pallas_call(
        _enc_edges_body,
        grid=(grid,),
        in_specs=[
            pl.BlockSpec((BLK_E, 128), lambda i: (i + blk0, 0)),
            pl.BlockSpec((128, 128), lambda i: (0, 0)),
            pl.BlockSpec((1, 128), lambda i: (0, 0)),
            pl.BlockSpec((128, 128), lambda i: (0, 0)),
            pl.BlockSpec((1, 128), lambda i: (0, 0)),
        ],
        out_specs=pl.BlockSpec((BLK_E, 128), lambda i: (i, 0)),
        out_shape=jax.ShapeDtypeStruct((CE8, 128), jnp.float32),
    )(edges8, _bd8(Wee1), _t8(bee1), _bd8(Wee2), _t8(bee2))


def _edge_step_body_full(e_ref, gs_ref, gr_ref, w1_ref, bh_ref, w2_ref, b2_ref,
                         enew_ref, elat_ref):
    e = e_ref[...]
    h = jnp.maximum(
        jnp.dot(e, w1_ref[...], preferred_element_type=jnp.float32)
        + gs_ref[...] + gr_ref[...] + bh_ref[...], 0.0)
    en = jnp.dot(h, w2_ref[...], preferred_element_type=jnp.float32) + b2_ref[...]
    enew_ref[...] = en
    elat_ref[...] = e + en


def _edge_step_body_last(e_ref, gs_ref, gr_ref, w1_ref, bh_ref, w2_ref, b2_ref,
                         enew_ref):
    e = e_ref[...]
    h = jnp.maximum(
        jnp.dot(e, w1_ref[...], preferred_element_type=jnp.float32)
        + gs_ref[...] + gr_ref[...] + bh_ref[...], 0.0)
    enew_ref[...] = (
        jnp.dot(h, w2_ref[...], preferred_element_type=jnp.float32) + b2_ref[...])


def _edge_step(e_lat8, Gs8, Gr8, W1e, bias_h, W2, b2, last):
    grid = CE8 // BLK_E
    n_out = 1 if last else 2
    return pl.pallas_call(
        _edge_step_body_last if last else _edge_step_body_full,
        grid=(grid,),
        in_specs=[
            pl.BlockSpec((BLK_E, 128), lambda i: (i, 0)),
            pl.BlockSpec((BLK_E, 128), lambda i: (i, 0)),
            pl.BlockSpec((BLK_E, 128), lambda i: (i, 0)),
            pl.BlockSpec((128, 128), lambda i: (0, 0)),
            pl.BlockSpec((1, 128), lambda i: (0, 0)),
            pl.BlockSpec((128, 128), lambda i: (0, 0)),
            pl.BlockSpec((1, 128), lambda i: (0, 0)),
        ],
        out_specs=[pl.BlockSpec((BLK_E, 128), lambda i: (i, 0))] * n_out,
        out_shape=[jax.ShapeDtypeStruct((CE8, 128), jnp.float32)] * n_out,
    )(e_lat8, Gs8, Gr8, _bd8(W1e), _t8(bias_h), _bd8(W2), _t8(b2))


def _node_step_body(n_ref, sa_ref, sb_ref, ra_ref, rb_ref,
                    wa_ref, wb_ref, wc_ref, bn_ref,
                    w2_ref, b2_ref, wx_ref, wy_ref, bx_ref,
                    nlat_ref, px_ref, py_ref):
    n = n_ref[...]
    s = (sa_ref[0] + sa_ref[1]) + (sb_ref[0] + sb_ref[1])
    r = (ra_ref[0] + ra_ref[1]) + (rb_ref[0] + rb_ref[1])
    h = jnp.maximum(
        jnp.dot(n, wa_ref[...], preferred_element_type=jnp.float32)
        + jnp.dot(s, wb_ref[...], preferred_element_type=jnp.float32)
        + jnp.dot(r, wc_ref[...], preferred_element_type=jnp.float32)
        + bn_ref[...], 0.0)
    nn = jnp.dot(h, w2_ref[...], preferred_element_type=jnp.float32) + b2_ref[...]
    nl = nn + n
    nlat_ref[...] = nl
    px_ref[...] = (
        jnp.dot(nl, wx_ref[...], preferred_element_type=jnp.float32) + bx_ref[...])
    py_ref[...] = jnp.dot(nl, wy_ref[...], preferred_element_type=jnp.float32)


def _node_step(n_lat8, sents, recvs, WA, WB, WC, bias_n, W2, b2, WX, WY, bx,
               last):
    if last:
        px_shape = jax.ShapeDtypeStruct((N8, 8 * D_FEAT), jnp.float32)
    else:
        px_shape = jax.ShapeDtypeStruct((N8, 128), jnp.float32)
    out_shape = [jax.ShapeDtypeStruct((N8, 128), jnp.float32), px_shape,
                 jax.ShapeDtypeStruct((N8, 128), jnp.float32)]
    return pl.pallas_call(
        _node_step_body,
        out_shape=out_shape,
    )(n_lat8, sents[0], sents[1], recvs[0], recvs[1],
      _bd8(WA), _bd8(WB), _bd8(WC), _t8(bias_n),
      _bd8(W2), _t8(b2), _bd8(WX), _bd8(WY), _t8(bx))


def _make_sc_gather(row0):
    @functools.partial(
        pl.kernel,
        out_type=[jax.ShapeDtypeStruct((CROWS, IDX_W, LATENT), jnp.float32)] * 2,
        mesh=_SC_MESH,
        compiler_params=_SC_PARAMS,
        scratch_types=[
            pltpu.VMEM((RPC, IDX_W), jnp.int32),
            pltpu.VMEM((RPC, IDX_W, LATENT), jnp.float32),
            pltpu.SemaphoreType.DMA,
        ],
    )
    def _sc_gather(ps_hbm, pr_hbm, snd_hbm, rcv_hbm, gs_hbm, gr_hbm,
                   idx_v, rows_v, sem):
        wid = lax.axis_index("s") * NC + lax.axis_index("c")
        for tbl, ih, oh in ((ps_hbm, snd_hbm, gs_hbm),
                            (pr_hbm, rcv_hbm, gr_hbm)):
            for t in range(ROUNDS):
                ob = wid * ROWS_PW + t * RPC
                pltpu.sync_copy(ih.at[pl.ds(row0 + ob, RPC)], idx_v)
                cps = [pltpu.async_copy(tbl.at[idx_v.at[j]], rows_v.at[j], sem)
                       for j in range(RPC)]
                for c in cps:
                    c.wait()
                pltpu.sync_copy(rows_v, oh.at[pl.ds(ob, RPC)])

            @pl.when(wid < TAIL)
            def _tail():
                ob = NW * ROWS_PW + wid
                pltpu.sync_copy(ih.at[pl.ds(row0 + ob, 1)],
                                idx_v.at[pl.ds(0, 1)])
                pltpu.async_copy(tbl.at[idx_v.at[0]], rows_v.at[0], sem).wait()
                pltpu.sync_copy(rows_v.at[pl.ds(0, 1)], oh.at[pl.ds(ob, 1)])

    return _sc_gather


def _make_sc_scatter(row0):
    @functools.partial(
        pl.kernel,
        out_type=[jax.ShapeDtypeStruct((NC, N, LATENT), jnp.float32)] * 2,
        mesh=_SC_MESH,
        compiler_params=_SC_PARAMS,
        scratch_types=[
            pltpu.VMEM((RPC, IDX_W), jnp.int32),
            pltpu.VMEM((RPC, IDX_W), jnp.int32),
            pltpu.VMEM((RPC, IDX_W, LATENT), jnp.float32),
            pltpu.VMEM_SHARED((N, LATENT), jnp.float32),
            pltpu.VMEM_SHARED((N, LATENT), jnp.float32),
            pltpu.SemaphoreType.DMA,
        ],
    )
    def _sc_scatter(enew_hbm, snd_hbm, rcv_hbm, zeros_hbm, sent_hbm, recv_hbm,
                    idx_s, idx_r, rows_v, acc_s, acc_r, sem):
        cid = lax.axis_index("c")
        sid = lax.axis_index("s")
        wid = sid * NC + cid

        @pl.when(sid == 0)
        def _zero():
            pltpu.sync_copy(zeros_hbm, acc_s)
            pltpu.sync_copy(zeros_hbm, acc_r)

        plsc.subcore_barrier()
        for t in range(ROUNDS):
            ob = wid * ROWS_PW + t * RPC
            pltpu.sync_copy(enew_hbm.at[pl.ds(ob, RPC)], rows_v)
            pltpu.sync_copy(snd_hbm.at[pl.ds(row0 + ob, RPC)], idx_s)
            pltpu.sync_copy(rcv_hbm.at[pl.ds(row0 + ob, RPC)], idx_r)
            cps = [pltpu.async_copy(rows_v.at[j], acc.at[ix.at[j]], sem,
                                    add=True)
                   for ix, acc in ((idx_s, acc_s), (idx_r, acc_r))
                   for j in range(RPC)]
            for c in cps:
                c.wait()

        @pl.when(wid < TAIL)
        def _tail():
            ob = NW * ROWS_PW + wid
            pltpu.sync_copy(enew_hbm.at[pl.ds(ob, 1)], rows_v.at[pl.ds(0, 1)])
            for ih, ix, acc in ((snd_hbm, idx_s, acc_s),
                                (rcv_hbm, idx_r, acc_r)):
                pltpu.sync_copy(ih.at[pl.ds(row0 + ob, 1)], ix.at[pl.ds(0, 1)])
                pltpu.sync_copy(rows_v.at[0], acc.at[ix.at[0]], add=True)

        plsc.subcore_barrier()

        @pl.when(sid == 0)
        def _writeout():
            pltpu.sync_copy(acc_s, sent_hbm.at[cid])
            pltpu.sync_copy(acc_r, recv_hbm.at[cid])

    return _sc_scatter


_SC_GATHERS = [_make_sc_gather(c * CROWS) for c in range(NCHUNK)]
_SC_SCATTERS = [_make_sc_scatter(c * CROWS) for c in range(NCHUNK)]


def _gather_proj(c, Ps8, Pr8, senders2d, receivers2d):
    gs, gr = _SC_GATHERS[c](Ps8.reshape(N, LATENT), Pr8.reshape(N, LATENT),
                            senders2d, receivers2d)
    return gs.reshape(CE8, 128), gr.reshape(CE8, 128)


def _segment_sums(c, e_new8, senders2d, receivers2d):
    e_new = e_new8.reshape(CROWS, IDX_W, LATENT)
    zeros = jnp.zeros((N, LATENT), jnp.float32)
    sent, recv = _SC_SCATTERS[c](e_new, senders2d, receivers2d, zeros)
    return sent.reshape(NC, N8, 128), recv.reshape(NC, N8, 128)


def kernel(nodes, edges, senders, receivers, aux_data, rng,
           Wen1, ben1, Wen2, ben2, Wee1, bee1, Wee2, bee2,
           We1, be1, We2, be2, Wn1, bn1, Wn2, bn2, Wdn, bdn, Wde, bde):
    g = aux_data
    senders2d = senders.reshape(IDX_ROWS, IDX_W)
    receivers2d = receivers.reshape(IDX_ROWS, IDX_W)

    n_lat8, Ps8, Pr8 = _enc_nodes(nodes, Wen1, ben1, Wen2, ben2,
                                  We1[0, 16:32], We1[0, 32:48])
    edges8 = edges.reshape(E8, 128)
    e_lat = [_enc_edges(c, edges8, Wee1, bee1, Wee2, bee2)
             for c in range(NCHUNK)]

    zeros16 = jnp.zeros((LATENT, LATENT), jnp.float32)
    for i in range(STEPS):
        bias_h = g @ We1[i, 48:56] + be1[i]
        last = i == STEPS - 1
        G = [_gather_proj(c, Ps8, Pr8, senders2d, receivers2d)
             for c in range(NCHUNK)]
        sents, recvs = [], []
        e_lat_new = []
        for c in range(NCHUNK):
            outs = _edge_step(e_lat[c], G[c][0], G[c][1], We1[i, 0:16],
                              bias_h, We2[i], be2[i], last)
            e_new8 = outs[0]
            if not last:
                e_lat_new.append(outs[1])
            s8, r8 = _segment_sums(c, e_new8, senders2d, receivers2d)
            sents.append(s8)
            recvs.append(r8)
        e_lat = e_lat_new
        bias_n = g @ Wn1[i, 48:56] + bn1[i]
        if last:
            WX, WY, bx = Wdn, zeros16, bdn
        else:
            WX, WY, bx = We1[i + 1, 16:32], We1[i + 1, 32:48], jnp.zeros(
                (LATENT,), jnp.float32)
        n_lat8, PX, PY = _node_step(n_lat8, sents, recvs,
                                    Wn1[i, 0:16], Wn1[i, 16:32], Wn1[i, 32:48],
                                    bias_n, Wn2[i], bn2[i], WX, WY, bx, last)
        if last:
            nodes_out = PX.reshape(N, D_FEAT)
        else:
            Ps8, Pr8 = PX, PY

    return nodes_out


# speedup vs baseline: 1.0563x; 1.0563x over previous
import functools

import jax
import jax.numpy as jnp
from jax import lax
from jax.experimental import pallas as pl
from jax.experimental.pallas import tpu as pltpu
from jax.experimental.pallas import tpu_sc as plsc

N = 10000
E = 320000
D_FEAT = 128
LATENT = 16
STEPS = 3

E8 = E // 8
N8 = N // 8
BLK_E = 2000

NC, NS = 2, 16
NW = NC * NS
IDX_W = 128
IDX_ROWS = E // IDX_W
NCHUNK = 1
CROWS = IDX_ROWS // NCHUNK
CE8 = E8 // NCHUNK
ROWS_PW = CROWS // NW
ROUNDS = 3
RPC = ROWS_PW // ROUNDS
TAIL = CROWS - NW * ROWS_PW

_SC_MESH = plsc.VectorSubcoreMesh(core_axis_name="c", subcore_axis_name="s")
_SC_PARAMS = pltpu.CompilerParams(use_tc_tiling_on_sc=False)


def _bd8(W):
    return jnp.kron(jnp.eye(8, dtype=W.dtype), W)


def _t8(b):
    return jnp.tile(b, 8)[None, :]


def _enc_nodes_body(x_ref, w1_ref, b1_ref, w2_ref, b2_ref, ws_ref, wr_ref,
                    nlat_ref, ps_ref, pr_ref):
    h = jnp.maximum(
        jnp.dot(x_ref[...], w1_ref[...], preferred_element_type=jnp.float32)
        + b1_ref[...], 0.0)
    nl = jnp.dot(h, w2_ref[...], preferred_element_type=jnp.float32) + b2_ref[...]
    nlat_ref[...] = nl
    ps_ref[...] = jnp.dot(nl, ws_ref[...], preferred_element_type=jnp.float32)
    pr_ref[...] = jnp.dot(nl, wr_ref[...], preferred_element_type=jnp.float32)


def _enc_nodes(nodes, Wen1, ben1, Wen2, ben2, Ws0, Wr0):
    return pl.pallas_call(
        _enc_nodes_body,
        out_shape=[jax.ShapeDtypeStruct((N8, 128), jnp.float32)] * 3,
    )(nodes.reshape(N8, 8 * D_FEAT), _bd8(Wen1), _t8(ben1), _bd8(Wen2),
      _t8(ben2), _bd8(Ws0), _bd8(Wr0))


def _enc_edges_body(x_ref, w1_ref, b1_ref, w2_ref, b2_ref, out_ref):
    h = jnp.maximum(
        jnp.dot(x_ref[...], w1_ref[...], preferred_element_type=jnp.float32)
        + b1_ref[...], 0.0)
    out_ref[...] = (
        jnp.dot(h, w2_ref[...], preferred_element_type=jnp.float32) + b2_ref[...])


def _enc_edges(c, edges8, Wee1, bee1, Wee2, bee2):
    grid = CE8 // BLK_E
    blk0 = c * grid
    return pl.pallas_call(
        _enc_edges_body,
        grid=(grid,),
        in_specs=[
            pl.BlockSpec((BLK_E, 128), lambda i: (i + blk0, 0)),
            pl.BlockSpec((128, 128), lambda i: (0, 0)),
            pl.BlockSpec((1, 128), lambda i: (0, 0)),
            pl.BlockSpec((128, 128), lambda i: (0, 0)),
            pl.BlockSpec((1, 128), lambda i: (0, 0)),
        ],
        out_specs=pl.BlockSpec((BLK_E, 128), lambda i: (i, 0)),
        out_shape=jax.ShapeDtypeStruct((CE8, 128), jnp.float32),
    )(edges8, _bd8(Wee1), _t8(bee1), _bd8(Wee2), _t8(bee2))


def _edge_step_body_full(e_ref, gs_ref, gr_ref, w1_ref, bh_ref, w2_ref, b2_ref,
                         enew_ref, elat_ref):
    e = e_ref[...]
    h = jnp.maximum(
        jnp.dot(e, w1_ref[...], preferred_element_type=jnp.float32)
        + gs_ref[...] + gr_ref[...] + bh_ref[...], 0.0)
    en = jnp.dot(h, w2_ref[...], preferred_element_type=jnp.float32) + b2_ref[...]
    enew_ref[...] = en
    elat_ref[...] = e + en


def _edge_step_body_last(e_ref, gs_ref, gr_ref, w1_ref, bh_ref, w2_ref, b2_ref,
                         enew_ref):
    e = e_ref[...]
    h = jnp.maximum(
        jnp.dot(e, w1_ref[...], preferred_element_type=jnp.float32)
        + gs_ref[...] + gr_ref[...] + bh_ref[...], 0.0)
    enew_ref[...] = (
        jnp.dot(h, w2_ref[...], preferred_element_type=jnp.float32) + b2_ref[...])


def _edge_step(e_lat8, Gs8, Gr8, W1e, bias_h, W2, b2, last):
    grid = CE8 // BLK_E
    n_out = 1 if last else 2
    return pl.pallas_call(
        _edge_step_body_last if last else _edge_step_body_full,
        grid=(grid,),
        in_specs=[
            pl.BlockSpec((BLK_E, 128), lambda i: (i, 0)),
            pl.BlockSpec((BLK_E, 128), lambda i: (i, 0)),
            pl.BlockSpec((BLK_E, 128), lambda i: (i, 0)),
            pl.BlockSpec((128, 128), lambda i: (0, 0)),
            pl.BlockSpec((1, 128), lambda i: (0, 0)),
            pl.BlockSpec((128, 128), lambda i: (0, 0)),
            pl.BlockSpec((1, 128), lambda i: (0, 0)),
        ],
        out_specs=[pl.BlockSpec((BLK_E, 128), lambda i: (i, 0))] * n_out,
        out_shape=[jax.ShapeDtypeStruct((CE8, 128), jnp.float32)] * n_out,
    )(e_lat8, Gs8, Gr8, _bd8(W1e), _t8(bias_h), _bd8(W2), _t8(b2))


def _node_step_body(*refs):
    (n_ref, *srefs), refs = refs[:1 + 2 * NCHUNK], refs[1 + 2 * NCHUNK:]
    (wa_ref, wb_ref, wc_ref, bn_ref, w2_ref, b2_ref, wx_ref, wy_ref, bx_ref,
     nlat_ref, px_ref, py_ref) = refs
    n = n_ref[...]
    parts = [p_ref[0] + p_ref[1] for p_ref in srefs]
    s = sum(parts[:NCHUNK])
    r = sum(parts[NCHUNK:])
    h = jnp.maximum(
        jnp.dot(n, wa_ref[...], preferred_element_type=jnp.float32)
        + jnp.dot(s, wb_ref[...], preferred_element_type=jnp.float32)
        + jnp.dot(r, wc_ref[...], preferred_element_type=jnp.float32)
        + bn_ref[...], 0.0)
    nn = jnp.dot(h, w2_ref[...], preferred_element_type=jnp.float32) + b2_ref[...]
    nl = nn + n
    nlat_ref[...] = nl
    px_ref[...] = (
        jnp.dot(nl, wx_ref[...], preferred_element_type=jnp.float32) + bx_ref[...])
    py_ref[...] = jnp.dot(nl, wy_ref[...], preferred_element_type=jnp.float32)


def _node_step(n_lat8, sents, recvs, WA, WB, WC, bias_n, W2, b2, WX, WY, bx,
               last):
    if last:
        px_shape = jax.ShapeDtypeStruct((N8, 8 * D_FEAT), jnp.float32)
    else:
        px_shape = jax.ShapeDtypeStruct((N8, 128), jnp.float32)
    out_shape = [jax.ShapeDtypeStruct((N8, 128), jnp.float32), px_shape,
                 jax.ShapeDtypeStruct((N8, 128), jnp.float32)]
    return pl.pallas_call(
        _node_step_body,
        out_shape=out_shape,
    )(n_lat8, *sents, *recvs,
      _bd8(WA), _bd8(WB), _bd8(WC), _t8(bias_n),
      _bd8(W2), _t8(b2), _bd8(WX), _bd8(WY), _t8(bx))


def _make_sc_gather(row0):
    @functools.partial(
        pl.kernel,
        out_type=[jax.ShapeDtypeStruct((CROWS, IDX_W, LATENT), jnp.float32)] * 2,
        mesh=_SC_MESH,
        compiler_params=_SC_PARAMS,
        scratch_types=[
            pltpu.VMEM((RPC, IDX_W), jnp.int32),
            pltpu.VMEM((RPC, IDX_W, LATENT), jnp.float32),
            pltpu.SemaphoreType.DMA,
        ],
    )
    def _sc_gather(ps_hbm, pr_hbm, snd_hbm, rcv_hbm, gs_hbm, gr_hbm,
                   idx_v, rows_v, sem):
        wid = lax.axis_index("s") * NC + lax.axis_index("c")
        for tbl, ih, oh in ((ps_hbm, snd_hbm, gs_hbm),
                            (pr_hbm, rcv_hbm, gr_hbm)):
            for t in range(ROUNDS):
                ob = wid * ROWS_PW + t * RPC
                pltpu.sync_copy(ih.at[pl.ds(row0 + ob, RPC)], idx_v)
                cps = [pltpu.async_copy(tbl.at[idx_v.at[j]], rows_v.at[j], sem)
                       for j in range(RPC)]
                for c in cps:
                    c.wait()
                pltpu.sync_copy(rows_v, oh.at[pl.ds(ob, RPC)])

            @pl.when(wid < TAIL)
            def _tail():
                ob = NW * ROWS_PW + wid
                pltpu.sync_copy(ih.at[pl.ds(row0 + ob, 1)],
                                idx_v.at[pl.ds(0, 1)])
                pltpu.async_copy(tbl.at[idx_v.at[0]], rows_v.at[0], sem).wait()
                pltpu.sync_copy(rows_v.at[pl.ds(0, 1)], oh.at[pl.ds(ob, 1)])

    return _sc_gather


def _make_sc_scatter(row0):
    @functools.partial(
        pl.kernel,
        out_type=[jax.ShapeDtypeStruct((NC, N, LATENT), jnp.float32)] * 2,
        mesh=_SC_MESH,
        compiler_params=_SC_PARAMS,
        scratch_types=[
            pltpu.VMEM((RPC, IDX_W), jnp.int32),
            pltpu.VMEM((RPC, IDX_W), jnp.int32),
            pltpu.VMEM((RPC, IDX_W, LATENT), jnp.float32),
            pltpu.VMEM_SHARED((N, LATENT), jnp.float32),
            pltpu.VMEM_SHARED((N, LATENT), jnp.float32),
            pltpu.SemaphoreType.DMA,
        ],
    )
    def _sc_scatter(enew_hbm, snd_hbm, rcv_hbm, zeros_hbm, sent_hbm, recv_hbm,
                    idx_s, idx_r, rows_v, acc_s, acc_r, sem):
        cid = lax.axis_index("c")
        sid = lax.axis_index("s")
        wid = sid * NC + cid

        @pl.when(sid == 0)
        def _zero():
            pltpu.sync_copy(zeros_hbm, acc_s)
            pltpu.sync_copy(zeros_hbm, acc_r)

        plsc.subcore_barrier()
        for t in range(ROUNDS):
            ob = wid * ROWS_PW + t * RPC
            pltpu.sync_copy(enew_hbm.at[pl.ds(ob, RPC)], rows_v)
            pltpu.sync_copy(snd_hbm.at[pl.ds(row0 + ob, RPC)], idx_s)
            pltpu.sync_copy(rcv_hbm.at[pl.ds(row0 + ob, RPC)], idx_r)
            cps = [pltpu.async_copy(rows_v.at[j], acc.at[ix.at[j]], sem,
                                    add=True)
                   for ix, acc in ((idx_s, acc_s), (idx_r, acc_r))
                   for j in range(RPC)]
            for c in cps:
                c.wait()

        @pl.when(wid < TAIL)
        def _tail():
            ob = NW * ROWS_PW + wid
            pltpu.sync_copy(enew_hbm.at[pl.ds(ob, 1)], rows_v.at[pl.ds(0, 1)])
            for ih, ix, acc in ((snd_hbm, idx_s, acc_s),
                                (rcv_hbm, idx_r, acc_r)):
                pltpu.sync_copy(ih.at[pl.ds(row0 + ob, 1)], ix.at[pl.ds(0, 1)])
                pltpu.sync_copy(rows_v.at[0], acc.at[ix.at[0]], add=True)

        plsc.subcore_barrier()

        @pl.when(sid == 0)
        def _writeout():
            pltpu.sync_copy(acc_s, sent_hbm.at[cid])
            pltpu.sync_copy(acc_r, recv_hbm.at[cid])

    return _sc_scatter


_SC_GATHERS = [_make_sc_gather(c * CROWS) for c in range(NCHUNK)]
_SC_SCATTERS = [_make_sc_scatter(c * CROWS) for c in range(NCHUNK)]


def _gather_proj(c, Ps8, Pr8, senders2d, receivers2d):
    gs, gr = _SC_GATHERS[c](Ps8.reshape(N, LATENT), Pr8.reshape(N, LATENT),
                            senders2d, receivers2d)
    return gs.reshape(CE8, 128), gr.reshape(CE8, 128)


def _segment_sums(c, e_new8, senders2d, receivers2d):
    e_new = e_new8.reshape(CROWS, IDX_W, LATENT)
    zeros = jnp.zeros((N, LATENT), jnp.float32)
    sent, recv = _SC_SCATTERS[c](e_new, senders2d, receivers2d, zeros)
    return sent.reshape(NC, N8, 128), recv.reshape(NC, N8, 128)


def kernel(nodes, edges, senders, receivers, aux_data, rng,
           Wen1, ben1, Wen2, ben2, Wee1, bee1, Wee2, bee2,
           We1, be1, We2, be2, Wn1, bn1, Wn2, bn2, Wdn, bdn, Wde, bde):
    g = aux_data
    senders2d = senders.reshape(IDX_ROWS, IDX_W)
    receivers2d = receivers.reshape(IDX_ROWS, IDX_W)

    n_lat8, Ps8, Pr8 = _enc_nodes(nodes, Wen1, ben1, Wen2, ben2,
                                  We1[0, 16:32], We1[0, 32:48])
    edges8 = edges.reshape(E8, 128)
    e_lat = [_enc_edges(c, edges8, Wee1, bee1, Wee2, bee2)
             for c in range(NCHUNK)]

    zeros16 = jnp.zeros((LATENT, LATENT), jnp.float32)
    for i in range(STEPS):
        bias_h = g @ We1[i, 48:56] + be1[i]
        last = i == STEPS - 1
        G = [_gather_proj(c, Ps8, Pr8, senders2d, receivers2d)
             for c in range(NCHUNK)]
        sents, recvs = [], []
        e_lat_new = []
        for c in range(NCHUNK):
            outs = _edge_step(e_lat[c], G[c][0], G[c][1], We1[i, 0:16],
                              bias_h, We2[i], be2[i], last)
            e_new8 = outs[0]
            if not last:
                e_lat_new.append(outs[1])
            s8, r8 = _segment_sums(c, e_new8, senders2d, receivers2d)
            sents.append(s8)
            recvs.append(r8)
        e_lat = e_lat_new
        bias_n = g @ Wn1[i, 48:56] + bn1[i]
        if last:
            WX, WY, bx = Wdn, zeros16, bdn
        else:
            WX, WY, bx = We1[i + 1, 16:32], We1[i + 1, 32:48], jnp.zeros(
                (LATENT,), jnp.float32)
        n_lat8, PX, PY = _node_step(n_lat8, sents, recvs,
                                    Wn1[i, 0:16], Wn1[i, 16:32], Wn1[i, 32:48],
                                    bias_n, Wn2[i], bn2[i], WX, WY, bx, last)
        if last:
            nodes_out = PX.reshape(N, D_FEAT)
        else:
            Ps8, Pr8 = PX, PY

    return nodes_out
